# Initial kernel scaffold; baseline (speedup 1.0000x reference)
import jax
import jax.numpy as jnp
from jax.experimental import pallas as pl


def kernel(x, edge_types, t, means_w, stds_w, mul_w, bias_w):
    raise NotImplementedError("write your pallas kernel here")


# speedup vs baseline: 1.9268x; 1.9268x over previous
import functools

import jax
import jax.numpy as jnp
from jax import lax
from jax.experimental import pallas as pl
from jax.experimental.pallas import tpu as pltpu
from jax.experimental.pallas import tpu_sc as plsc

_N = 320000
_K = 128
_T = 5000
_E = 1024
_CH = 128
_NCH = _N // _CH
_NW = 32
_NC = 2
_INV_SQRT_2PI = 1.0 / (2.0 * 3.14159) ** 0.5


def _prep_body(m_ref, s_ref, o_ref):
    o_ref[:, :_K] = m_ref[...]
    o_ref[:, _K:] = 1.0 / (jnp.abs(s_ref[...]) + 1e-5)


def _pack_tables(means_w, stds_w):
    rows = 1000
    return pl.pallas_call(
        _prep_body,
        grid=(_T // rows,),
        in_specs=[
            pl.BlockSpec((rows, _K), lambda i: (i, 0)),
            pl.BlockSpec((rows, _K), lambda i: (i, 0)),
        ],
        out_specs=pl.BlockSpec((rows, 2 * _K), lambda i: (i, 0)),
        out_shape=jax.ShapeDtypeStruct((_T, 2 * _K), jnp.float32),
    )(means_w, stds_w)


@functools.partial(
    pl.kernel,
    out_type=jax.ShapeDtypeStruct((_N, _K), jnp.float32),
    mesh=plsc.VectorSubcoreMesh(core_axis_name="c", subcore_axis_name="s"),
    scratch_types=[
        pltpu.VMEM((_CH,), jnp.int32),
        pltpu.VMEM((_CH,), jnp.int32),
        pltpu.VMEM((_CH,), jnp.float32),
        pltpu.VMEM((_CH,), jnp.float32),
        pltpu.VMEM((_CH,), jnp.float32),
        pltpu.VMEM((_CH, 2 * _K), jnp.float32),
        pltpu.VMEM((_CH, _K), jnp.float32),
        pltpu.SemaphoreType.DMA,
    ],
)
def _sc_main(x_hbm, t_hbm, et_hbm, tbl_hbm, mul_hbm, bias_hbm, out_hbm,
             tidx_v, eidx_v, x_v, mul_v, bias_v, rows_v, out_v, sem):
    wid = lax.axis_index("s") * _NC + lax.axis_index("c")
    n_mine = (_NCH - wid + _NW - 1) // _NW

    def chunk_body(i, carry):
        base = (wid + i * _NW) * _CH
        pltpu.sync_copy(t_hbm.at[pl.ds(base, _CH)], tidx_v)
        pltpu.sync_copy(et_hbm.at[pl.ds(base, _CH)], eidx_v)
        pltpu.sync_copy(x_hbm.at[pl.ds(base, _CH)], x_v)
        cp_rows = pltpu.async_copy(tbl_hbm.at[tidx_v], rows_v, sem)
        cp_mul = pltpu.async_copy(mul_hbm.at[eidx_v], mul_v, sem)
        cp_bias = pltpu.async_copy(bias_hbm.at[eidx_v], bias_v, sem)
        cp_rows.wait()
        cp_mul.wait()
        cp_bias.wait()

        def group_body(g, c2):
            b16 = g * 16
            xe = (mul_v[pl.ds(b16, 16)] * x_v[pl.ds(b16, 16)]
                  + bias_v[pl.ds(b16, 16)])
            for rl in range(16):
                r = b16 + rl
                xr = xe[rl]
                for j in range(_K // 16):
                    m = rows_v[r, pl.ds(16 * j, 16)]
                    inv = rows_v[r, pl.ds(_K + 16 * j, 16)]
                    z = (xr - m) * inv
                    out_v[r, pl.ds(16 * j, 16)] = (
                        jnp.exp(z * z * (-0.5)) * inv * _INV_SQRT_2PI
                    )
            return c2

        lax.fori_loop(0, _CH // 16, group_body, 0)
        pltpu.sync_copy(out_v, out_hbm.at[pl.ds(base, _CH)])
        return carry

    lax.fori_loop(0, n_mine, chunk_body, 0)


def kernel(x, edge_types, t, means_w, stds_w, mul_w, bias_w):
    tbl = _pack_tables(means_w, stds_w)
    out = _sc_main(
        x.astype(jnp.float32),
        t.astype(jnp.int32),
        edge_types.astype(jnp.int32),
        tbl,
        mul_w.reshape(_E).astype(jnp.float32),
        bias_w.reshape(_E).astype(jnp.float32),
    )
    return out.astype(means_w.dtype)
